# baseline (speedup 1.0000x reference)
import functools

import jax
import jax.numpy as jnp
from jax import lax
from jax.experimental import pallas as pl
from jax.experimental.pallas import tpu as pltpu
from jax.experimental.pallas import tpu_sc as plsc

N = 50000
NPAD = 50048
S = NPAD // 16
E = 800000
EPAD = 802816
EB = 128
G = 512
BLK = 3128
NBLK = NPAD // BLK

_MESH = plsc.VectorSubcoreMesh(
    core_axis_name="c", subcore_axis_name="s", num_cores=2, num_subcores=16
)
_SC_PARAMS = pltpu.CompilerParams(use_tc_tiling_on_sc=False)


def _deg_call(eidx, z16, ones16, kg=2):
    nbt = EPAD // EB
    nb = nbt // 32
    ng = nb // kg
    assert ng % 2 == 0 and ng >= 4

    @functools.partial(
        pl.kernel,
        out_type=jax.ShapeDtypeStruct((4, NPAD, 16), jnp.float32),
        mesh=_MESH,
        compiler_params=_SC_PARAMS,
        scratch_types=[
            pltpu.VMEM((2, kg, 2, EB), jnp.int32),
            pltpu.VMEM((2, kg, EB), jnp.int32),
            pltpu.VMEM((2, kg, EB), jnp.int32),
            pltpu.VMEM((EB, 16), jnp.float32),
            pltpu.VMEM_SHARED((NPAD, 16), jnp.float32),
            pltpu.VMEM_SHARED((NPAD, 16), jnp.float32),
            pltpu.SemaphoreType.DMA((2,)),
            pltpu.SemaphoreType.DMA((2, kg)),
            pltpu.SemaphoreType.DMA((2, kg)),
        ],
    )
    def deg_kernel(eidx_hbm, z_hbm, ones_hbm, out_hbm,
                   slab_v, sidx_v, ridx_v, ones_v, accs_sh, accr_sh,
                   isem, s1sem, s2sem):
        core = lax.axis_index("c")
        sub = lax.axis_index("s")
        base_row = sub * S
        pltpu.sync_copy(ones_hbm, ones_v)
        pltpu.sync_copy(z_hbm.at[pl.ds(base_row, S)],
                        accs_sh.at[pl.ds(base_row, S)])
        pltpu.sync_copy(z_hbm.at[pl.ds(base_row, S)],
                        accr_sh.at[pl.ds(base_row, S)])
        plsc.subcore_barrier()
        gbase = (core * 16 + sub) * ng

        def slab_copy(gblk, p):
            return pltpu.make_async_copy(
                eidx_hbm.at[pl.ds(gblk * kg, kg)], slab_v.at[p], isem.at[p])

        def scat1(p, k):
            return pltpu.make_async_copy(
                ones_v, accs_sh.at[sidx_v.at[p, k]], s1sem.at[p, k])

        def scat2(p, k):
            return pltpu.make_async_copy(
                ones_v, accr_sh.at[ridx_v.at[p, k]], s2sem.at[p, k])

        def run_group(g, p):
            slab_copy(gbase + g, p).wait()
            for k in range(kg):
                @pl.when(g >= 2)
                def _():
                    scat1(p, k).wait()
                    scat2(p, k).wait()
                for v in range(EB // 16):
                    sl = pl.ds(v * 16, 16)
                    sidx_v[p, k, sl] = slab_v[p, k, 0, sl]
                    ridx_v[p, k, sl] = slab_v[p, k, 1, sl]
                pltpu.async_copy(ones_v, accs_sh.at[sidx_v.at[p, k]],
                                 s1sem.at[p, k], add=True)
                pltpu.async_copy(ones_v, accr_sh.at[ridx_v.at[p, k]],
                                 s2sem.at[p, k], add=True)
            @pl.when(g + 1 < ng)
            def _():
                slab_copy(gbase + g + 1, 1 - p).start()

        slab_copy(gbase, 0).start()

        def pair(t, carry):
            run_group(2 * t, 0)
            run_group(2 * t + 1, 1)
            return carry

        lax.fori_loop(0, ng // 2, pair, 0)
        for p in range(2):
            for k in range(kg):
                scat1(p, k).wait()
                scat2(p, k).wait()
        plsc.subcore_barrier()
        pltpu.sync_copy(accs_sh.at[pl.ds(base_row, S)],
                        out_hbm.at[core, pl.ds(base_row, S)])
        pltpu.sync_copy(accr_sh.at[pl.ds(base_row, S)],
                        out_hbm.at[core + 2, pl.ds(base_row, S)])

    return deg_kernel(eidx, z16, ones16)


def _prop_call(h_flat, eidx, z, ncc, nes, dc, kg):
    nunits = ncc * nes
    upc = nunits // 2
    nbt = EPAD // EB
    nb = nbt // (nes * 16)
    ng = nb // kg
    assert ng % 2 == 0 and ng >= 4

    if nes == 1:
        out_type = jax.ShapeDtypeStruct((NPAD, nunits, dc), jnp.float32)
    else:
        out_type = jax.ShapeDtypeStruct((nunits, NPAD, dc), jnp.float32)

    @functools.partial(
        pl.kernel,
        out_type=out_type,
        mesh=_MESH,
        compiler_params=_SC_PARAMS,
        scratch_types=[
            pltpu.VMEM((2, kg, 2, EB), jnp.int32),
            pltpu.VMEM((2, kg, EB), jnp.int32),
            pltpu.VMEM((2, kg, EB), jnp.int32),
            pltpu.VMEM((2, kg, EB, dc), jnp.float32),
            pltpu.VMEM_SHARED((NPAD, dc), jnp.float32),
            pltpu.SemaphoreType.DMA((2,)),
            pltpu.SemaphoreType.DMA((2, kg)),
            pltpu.SemaphoreType.DMA((2, kg)),
        ],
    )
    def prop_kernel(h_hbm, eidx_hbm, z_hbm, out_hbm,
                    slab_v, sidx2_v, ridx_v, rows_v, acc_sh,
                    isem, gsem, ssem):
        core = lax.axis_index("c")
        sub = lax.axis_index("s")
        base_row = sub * S

        def slab_copy(gblk, p):
            return pltpu.make_async_copy(
                eidx_hbm.at[pl.ds(gblk * kg, kg)], slab_v.at[p], isem.at[p])

        def gather_copy(p, k):
            return pltpu.make_async_copy(
                h_hbm.at[sidx2_v.at[p, k]], rows_v.at[p, k], gsem.at[p, k])

        def scatter_copy(p, k):
            return pltpu.make_async_copy(
                rows_v.at[p, k], acc_sh.at[ridx_v.at[p, k]], ssem.at[p, k])

        for j in range(upc):
            u = 2 * j + core
            if nes == 1:
                cc, es = u, 0
            else:
                cc, es = 0, u
            pltpu.sync_copy(z_hbm.at[pl.ds(base_row, S)],
                            acc_sh.at[pl.ds(base_row, S)])
            plsc.subcore_barrier()
            gbase = es * (nbt // (nes * kg)) + sub * ng

            def run_group(g, p):
                slab_copy(gbase + g, p).wait()
                for k in range(kg):
                    @pl.when(g >= 2)
                    def _():
                        scatter_copy(p, k).wait()
                    for v in range(EB // 16):
                        sl = pl.ds(v * 16, 16)
                        sidx2_v[p, k, sl] = slab_v[p, k, 0, sl] * ncc + cc
                        ridx_v[p, k, sl] = slab_v[p, k, 1, sl]
                    pltpu.async_copy(
                        h_hbm.at[sidx2_v.at[p, k]], rows_v.at[p, k],
                        gsem.at[p, k])
                @pl.when(g + 1 < ng)
                def _():
                    slab_copy(gbase + g + 1, 1 - p).start()
                for k in range(kg):
                    gather_copy(p, k).wait()
                    pltpu.async_copy(
                        rows_v.at[p, k], acc_sh.at[ridx_v.at[p, k]],
                        ssem.at[p, k], add=True)

            slab_copy(gbase, 0).start()

            def pair(t, carry):
                run_group(2 * t, 0)
                run_group(2 * t + 1, 1)
                return carry

            lax.fori_loop(0, ng // 2, pair, 0)
            for p in range(2):
                for k in range(kg):
                    scatter_copy(p, k).wait()
            plsc.subcore_barrier()
            if nes == 1:
                pltpu.sync_copy(acc_sh.at[pl.ds(base_row, S)],
                                out_hbm.at[pl.ds(base_row, S), u])
            else:
                pltpu.sync_copy(acc_sh.at[pl.ds(base_row, S)],
                                out_hbm.at[u, pl.ds(base_row, S)])

    return prop_kernel(h_flat, eidx, z)


def _tc_layer(prev, degs, wmat, bvec, relu_in):
    _, din = prev.shape
    dout = wmat.shape[1]

    def body(prev_ref, degs_ref, w_ref, b_ref, out_ref):
        i = pl.program_id(0)
        rows = i * BLK + lax.broadcasted_iota(jnp.int32, (BLK, 16), 0)
        live = rows < N
        dego = degs_ref[0] + degs_ref[1]
        inv_out = jnp.where(
            live, lax.rsqrt(jnp.maximum(dego, 1.0)), 0.0)[:, 0:1]
        t = prev_ref[...]
        if relu_in:
            degi = degs_ref[2] + degs_ref[3]
            inv_in = lax.rsqrt(jnp.maximum(degi, 1.0))[:, 0:1]
            t = jnp.maximum(t * inv_in, 0.0)
        h = jnp.dot(t, w_ref[...], preferred_element_type=jnp.float32)
        out_ref[...] = (h + b_ref[...]) * inv_out

    return pl.pallas_call(
        body,
        grid=(NBLK,),
        in_specs=[
            pl.BlockSpec((BLK, din), lambda i: (i, 0)),
            pl.BlockSpec((4, BLK, 16), lambda i: (0, i, 0)),
            pl.BlockSpec((din, dout), lambda i: (0, 0)),
            pl.BlockSpec((1, dout), lambda i: (0, 0)),
        ],
        out_specs=pl.BlockSpec((BLK, dout), lambda i: (i, 0)),
        out_shape=jax.ShapeDtypeStruct((NPAD, dout), jnp.float32),
    )(prev, degs, wmat, bvec)


def _tc_readout(parts, degs, batchw):
    def body(parts_ref, degs_ref, batch_ref, out_ref):
        i = pl.program_id(0)

        @pl.when(i == 0)
        def _():
            out_ref[...] = jnp.zeros_like(out_ref)

        degi = degs_ref[2] + degs_ref[3]
        inv_in = lax.rsqrt(jnp.maximum(degi, 1.0))[:, 0:1]
        h = (parts_ref[0] + parts_ref[1]) * inv_in
        bcol = batch_ref[...][:, 0:1]
        gids = lax.broadcasted_iota(jnp.int32, (BLK, G), 1)
        mask = jnp.where(bcol == gids, 1.0, 0.0)
        out_ref[...] += lax.dot_general(
            mask, h, (((0,), (0,)), ((), ())),
            preferred_element_type=jnp.float32,
        )

    return pl.pallas_call(
        body,
        grid=(NBLK,),
        in_specs=[
            pl.BlockSpec((2, BLK, 16), lambda i: (0, i, 0)),
            pl.BlockSpec((4, BLK, 16), lambda i: (0, i, 0)),
            pl.BlockSpec((BLK, 16), lambda i: (i, 0)),
        ],
        out_specs=pl.BlockSpec((G, 16), lambda i: (0, 0)),
        out_shape=jax.ShapeDtypeStruct((G, 16), jnp.float32),
    )(parts, degs, batchw)


def kernel(x, senders, receivers, batch, n_node, num_graphs,
           W1, b1, W2, b2, W3, b3):
    xp = jnp.pad(x, ((0, NPAD - N), (0, 16 - x.shape[1])))
    sp = jnp.pad(senders, (0, EPAD - E), constant_values=N)
    rp = jnp.pad(receivers, (0, EPAD - E), constant_values=N)
    bpw = jnp.broadcast_to(
        jnp.pad(batch, (0, NPAD - N)).reshape(NPAD, 1), (NPAD, 16))
    z32 = jnp.zeros((NPAD, 32), jnp.float32)
    z16 = jnp.zeros((NPAD, 16), jnp.float32)
    ones16 = jnp.ones((EB, 16), jnp.float32)
    w1p = jnp.pad(W1, ((0, 16 - W1.shape[0]), (0, 0)))
    w3p = jnp.pad(W3, ((0, 0), (0, 16 - W3.shape[1])))
    b1r = b1.reshape(1, -1)
    b2r = b2.reshape(1, -1)
    b3r = jnp.pad(b3, (0, 16 - b3.shape[0])).reshape(1, -1)

    eidx = jnp.stack([sp.reshape(-1, EB), rp.reshape(-1, EB)], axis=1)

    degs = _deg_call(eidx, z16, ones16)

    h1 = _tc_layer(xp, degs, w1p, b1r, False)
    a1 = _prop_call(h1.reshape(2 * NPAD, 32), eidx, z32, 2, 1, 32, 2)
    h2 = _tc_layer(a1.reshape(NPAD, 64), degs, W2, b2r, True)
    a2 = _prop_call(h2.reshape(4 * NPAD, 32), eidx, z32, 4, 1, 32, 2)
    h3 = _tc_layer(a2.reshape(NPAD, 128), degs, w3p, b3r, True)
    a3 = _prop_call(h3, eidx, z16, 1, 2, 16, 2)
    out = _tc_readout(a3, degs, bpw)
    return out[:, :10]


# speedup vs baseline: 1.1262x; 1.1262x over previous
import functools

import jax
import jax.numpy as jnp
from jax import lax
from jax.experimental import pallas as pl
from jax.experimental.pallas import tpu as pltpu
from jax.experimental.pallas import tpu_sc as plsc

N = 50000
NPAD = 50048
S = NPAD // 16
E = 800000
EPAD = 802816
EB = 128
G = 512
BLK = 3128
NBLK = NPAD // BLK

_MESH = plsc.VectorSubcoreMesh(
    core_axis_name="c", subcore_axis_name="s", num_cores=2, num_subcores=16
)
_SC_PARAMS = pltpu.CompilerParams(use_tc_tiling_on_sc=False)


def _deg_call(sblk, rblk, z16, ones16, kg=2):
    nbt = EPAD // EB
    nb = nbt // 32
    ng = nb // kg
    assert ng % 2 == 0 and ng >= 4

    @functools.partial(
        pl.kernel,
        out_type=jax.ShapeDtypeStruct((4, NPAD, 16), jnp.float32),
        mesh=_MESH,
        compiler_params=_SC_PARAMS,
        scratch_types=[
            pltpu.VMEM((2, kg, EB), jnp.int32),
            pltpu.VMEM((2, kg, EB), jnp.int32),
            pltpu.VMEM((2, kg, EB), jnp.int32),
            pltpu.VMEM((2, kg, EB), jnp.int32),
            pltpu.VMEM((EB, 16), jnp.float32),
            pltpu.VMEM_SHARED((NPAD, 16), jnp.float32),
            pltpu.VMEM_SHARED((NPAD, 16), jnp.float32),
            pltpu.SemaphoreType.DMA((2,)),
            pltpu.SemaphoreType.DMA((2,)),
            pltpu.SemaphoreType.DMA((2, kg)),
            pltpu.SemaphoreType.DMA((2, kg)),
        ],
    )
    def deg_kernel(s_hbm, r_hbm, z_hbm, ones_hbm, out_hbm,
                   slabs_v, slabr_v, sidx_v, ridx_v, ones_v,
                   accs_sh, accr_sh, isems, isemr, s1sem, s2sem):
        core = lax.axis_index("c")
        sub = lax.axis_index("s")
        base_row = sub * S
        pltpu.sync_copy(ones_hbm, ones_v)
        pltpu.sync_copy(z_hbm.at[pl.ds(base_row, S)],
                        accs_sh.at[pl.ds(base_row, S)])
        pltpu.sync_copy(z_hbm.at[pl.ds(base_row, S)],
                        accr_sh.at[pl.ds(base_row, S)])
        plsc.subcore_barrier()
        gbase = (core * 16 + sub) * ng

        def slab_copy_s(gblk, p):
            return pltpu.make_async_copy(
                s_hbm.at[pl.ds(gblk * kg, kg)], slabs_v.at[p], isems.at[p])

        def slab_copy_r(gblk, p):
            return pltpu.make_async_copy(
                r_hbm.at[pl.ds(gblk * kg, kg)], slabr_v.at[p], isemr.at[p])

        def scat1(p, k):
            return pltpu.make_async_copy(
                ones_v, accs_sh.at[sidx_v.at[p, k]], s1sem.at[p, k])

        def scat2(p, k):
            return pltpu.make_async_copy(
                ones_v, accr_sh.at[ridx_v.at[p, k]], s2sem.at[p, k])

        def run_group(g, p):
            slab_copy_s(gbase + g, p).wait()
            slab_copy_r(gbase + g, p).wait()
            for k in range(kg):
                @pl.when(g >= 2)
                def _():
                    scat1(p, k).wait()
                    scat2(p, k).wait()
                for v in range(EB // 16):
                    sl = pl.ds(v * 16, 16)
                    sidx_v[p, k, sl] = slabs_v[p, k, sl]
                    ridx_v[p, k, sl] = slabr_v[p, k, sl]
                pltpu.async_copy(ones_v, accs_sh.at[sidx_v.at[p, k]],
                                 s1sem.at[p, k], add=True)
                pltpu.async_copy(ones_v, accr_sh.at[ridx_v.at[p, k]],
                                 s2sem.at[p, k], add=True)
            @pl.when(g + 1 < ng)
            def _():
                slab_copy_s(gbase + g + 1, 1 - p).start()
                slab_copy_r(gbase + g + 1, 1 - p).start()

        slab_copy_s(gbase, 0).start()
        slab_copy_r(gbase, 0).start()

        def pair(t, carry):
            run_group(2 * t, 0)
            run_group(2 * t + 1, 1)
            return carry

        lax.fori_loop(0, ng // 2, pair, 0)
        for p in range(2):
            for k in range(kg):
                scat1(p, k).wait()
                scat2(p, k).wait()
        plsc.subcore_barrier()
        pltpu.sync_copy(accs_sh.at[pl.ds(base_row, S)],
                        out_hbm.at[core, pl.ds(base_row, S)])
        pltpu.sync_copy(accr_sh.at[pl.ds(base_row, S)],
                        out_hbm.at[core + 2, pl.ds(base_row, S)])

    return deg_kernel(sblk, rblk, z16, ones16)


def _prop_call(h_flat, sblk, rblk, z, ncc, nes, dc, kg):
    nunits = ncc * nes
    upc = nunits // 2
    nbt = EPAD // EB
    nb = nbt // (nes * 16)
    ng = nb // kg
    assert ng % 2 == 0 and ng >= 4

    @functools.partial(
        pl.kernel,
        out_type=jax.ShapeDtypeStruct((nunits, NPAD, dc), jnp.float32),
        mesh=_MESH,
        compiler_params=_SC_PARAMS,
        scratch_types=[
            pltpu.VMEM((2, kg, EB), jnp.int32),
            pltpu.VMEM((2, kg, EB), jnp.int32),
            pltpu.VMEM((2, kg, EB), jnp.int32),
            pltpu.VMEM((2, kg, EB), jnp.int32),
            pltpu.VMEM((2, kg, EB, dc), jnp.float32),
            pltpu.VMEM_SHARED((NPAD, dc), jnp.float32),
            pltpu.SemaphoreType.DMA((2,)),
            pltpu.SemaphoreType.DMA((2,)),
            pltpu.SemaphoreType.DMA((2, kg)),
            pltpu.SemaphoreType.DMA((2, kg)),
        ],
    )
    def prop_kernel(h_hbm, s_hbm, r_hbm, z_hbm, out_hbm,
                    slabs_v, slabr_v, sidx2_v, ridx_v, rows_v, acc_sh,
                    isems, isemr, gsem, ssem):
        core = lax.axis_index("c")
        sub = lax.axis_index("s")
        base_row = sub * S

        def slab_copy_s(gblk, p):
            return pltpu.make_async_copy(
                s_hbm.at[pl.ds(gblk * kg, kg)], slabs_v.at[p], isems.at[p])

        def slab_copy_r(gblk, p):
            return pltpu.make_async_copy(
                r_hbm.at[pl.ds(gblk * kg, kg)], slabr_v.at[p], isemr.at[p])

        def gather_copy(p, k):
            return pltpu.make_async_copy(
                h_hbm.at[sidx2_v.at[p, k]], rows_v.at[p, k], gsem.at[p, k])

        def scatter_copy(p, k):
            return pltpu.make_async_copy(
                rows_v.at[p, k], acc_sh.at[ridx_v.at[p, k]], ssem.at[p, k])

        for j in range(upc):
            u = 2 * j + core
            if nes == 1:
                cc, es = u, 0
            else:
                cc, es = 0, u
            pltpu.sync_copy(z_hbm.at[pl.ds(base_row, S)],
                            acc_sh.at[pl.ds(base_row, S)])
            plsc.subcore_barrier()
            gbase = es * (nbt // (nes * kg)) + sub * ng

            def run_group(g, p):
                slab_copy_s(gbase + g, p).wait()
                slab_copy_r(gbase + g, p).wait()
                for k in range(kg):
                    @pl.when(g >= 2)
                    def _():
                        scatter_copy(p, k).wait()
                    for v in range(EB // 16):
                        sl = pl.ds(v * 16, 16)
                        sidx2_v[p, k, sl] = slabs_v[p, k, sl] * ncc + cc
                        ridx_v[p, k, sl] = slabr_v[p, k, sl]
                    pltpu.async_copy(
                        h_hbm.at[sidx2_v.at[p, k]], rows_v.at[p, k],
                        gsem.at[p, k])
                @pl.when(g + 1 < ng)
                def _():
                    slab_copy_s(gbase + g + 1, 1 - p).start()
                    slab_copy_r(gbase + g + 1, 1 - p).start()
                for k in range(kg):
                    gather_copy(p, k).wait()
                    pltpu.async_copy(
                        rows_v.at[p, k], acc_sh.at[ridx_v.at[p, k]],
                        ssem.at[p, k], add=True)

            slab_copy_s(gbase, 0).start()
            slab_copy_r(gbase, 0).start()

            def pair(t, carry):
                run_group(2 * t, 0)
                run_group(2 * t + 1, 1)
                return carry

            lax.fori_loop(0, ng // 2, pair, 0)
            for p in range(2):
                for k in range(kg):
                    scatter_copy(p, k).wait()
            plsc.subcore_barrier()
            pltpu.sync_copy(acc_sh.at[pl.ds(base_row, S)],
                            out_hbm.at[u, pl.ds(base_row, S)])

    return prop_kernel(h_flat, sblk, rblk, z)


def _tc_layer(prev_ch, degs, wmat, bvec, relu_in):
    ncc_in, _, dc_in = prev_ch.shape
    din, dout = wmat.shape

    def body(prev_ref, degs_ref, w_ref, b_ref, out_ref):
        i = pl.program_id(0)
        rows = i * BLK + lax.broadcasted_iota(jnp.int32, (BLK, 16), 0)
        dego = degs_ref[0] + degs_ref[1]
        inv_out = jnp.where(
            rows < N, lax.rsqrt(jnp.maximum(dego, 1.0)), 0.0)[:, 0:1]
        if relu_in:
            degi = degs_ref[2] + degs_ref[3]
            inv_in = lax.rsqrt(jnp.maximum(degi, 1.0))[:, 0:1]
        h = jnp.zeros((BLK, dout), jnp.float32)
        for c in range(ncc_in):
            t = prev_ref[c]
            if relu_in:
                t = jnp.maximum(t * inv_in, 0.0)
            h = h + jnp.dot(t, w_ref[c * dc_in:(c + 1) * dc_in, :],
                            preferred_element_type=jnp.float32)
        out_ref[...] = (h + b_ref[...]) * inv_out

    return pl.pallas_call(
        body,
        grid=(NBLK,),
        in_specs=[
            pl.BlockSpec((ncc_in, BLK, dc_in), lambda i: (0, i, 0)),
            pl.BlockSpec((4, BLK, 16), lambda i: (0, i, 0)),
            pl.BlockSpec((din, dout), lambda i: (0, 0)),
            pl.BlockSpec((1, dout), lambda i: (0, 0)),
        ],
        out_specs=pl.BlockSpec((BLK, dout), lambda i: (i, 0)),
        out_shape=jax.ShapeDtypeStruct((NPAD, dout), jnp.float32),
    )(prev_ch, degs, wmat, bvec)


def _tc_readout(parts, degs, batchw):
    def body(parts_ref, degs_ref, batch_ref, out_ref):
        i = pl.program_id(0)

        @pl.when(i == 0)
        def _():
            out_ref[...] = jnp.zeros_like(out_ref)

        degi = degs_ref[2] + degs_ref[3]
        inv_in = lax.rsqrt(jnp.maximum(degi, 1.0))[:, 0:1]
        h = (parts_ref[0] + parts_ref[1]) * inv_in
        bcol = batch_ref[...][:, 0:1]
        gids = lax.broadcasted_iota(jnp.int32, (BLK, G), 1)
        mask = jnp.where(bcol == gids, 1.0, 0.0)
        out_ref[...] += lax.dot_general(
            mask, h, (((0,), (0,)), ((), ())),
            preferred_element_type=jnp.float32,
        )

    return pl.pallas_call(
        body,
        grid=(NBLK,),
        in_specs=[
            pl.BlockSpec((2, BLK, 16), lambda i: (0, i, 0)),
            pl.BlockSpec((4, BLK, 16), lambda i: (0, i, 0)),
            pl.BlockSpec((BLK, 16), lambda i: (i, 0)),
        ],
        out_specs=pl.BlockSpec((G, 16), lambda i: (0, 0)),
        out_shape=jax.ShapeDtypeStruct((G, 16), jnp.float32),
    )(parts, degs, batchw)


def kernel(x, senders, receivers, batch, n_node, num_graphs,
           W1, b1, W2, b2, W3, b3):
    xp = jnp.pad(x, ((0, NPAD - N), (0, 16 - x.shape[1])))
    sp = jnp.pad(senders, (0, EPAD - E), constant_values=N)
    rp = jnp.pad(receivers, (0, EPAD - E), constant_values=N)
    bpw = jnp.broadcast_to(
        jnp.pad(batch, (0, NPAD - N)).reshape(NPAD, 1), (NPAD, 16))
    z32 = jnp.zeros((NPAD, 32), jnp.float32)
    z16 = jnp.zeros((NPAD, 16), jnp.float32)
    ones16 = jnp.ones((EB, 16), jnp.float32)
    w1p = jnp.pad(W1, ((0, 16 - W1.shape[0]), (0, 0)))
    w3p = jnp.pad(W3, ((0, 0), (0, 16 - W3.shape[1])))
    b1r = b1.reshape(1, -1)
    b2r = b2.reshape(1, -1)
    b3r = jnp.pad(b3, (0, 16 - b3.shape[0])).reshape(1, -1)

    sblk = sp.reshape(-1, EB)
    rblk = rp.reshape(-1, EB)

    degs = _deg_call(sblk, rblk, z16, ones16)

    h1 = _tc_layer(xp.reshape(1, NPAD, 16), degs, w1p, b1r, False)
    a1 = _prop_call(h1.reshape(2 * NPAD, 32), sblk, rblk, z32, 2, 1, 32, 2)
    h2 = _tc_layer(a1, degs, W2, b2r, True)
    a2 = _prop_call(h2.reshape(4 * NPAD, 32), sblk, rblk, z32, 4, 1, 32, 2)
    h3 = _tc_layer(a2, degs, w3p, b3r, True)
    a3 = _prop_call(h3, sblk, rblk, z16, 1, 2, 16, 2)
    out = _tc_readout(a3, degs, bpw)
    return out[:, :10]


# speedup vs baseline: 1.1529x; 1.0236x over previous
import functools

import jax
import jax.numpy as jnp
from jax import lax
from jax.experimental import pallas as pl
from jax.experimental.pallas import tpu as pltpu
from jax.experimental.pallas import tpu_sc as plsc

N = 50000
NPAD = 50048
S = NPAD // 16
E = 800000
EPAD = 802816
EB = 128
G = 512
BLK = 2944
NBLK = NPAD // BLK

_MESH = plsc.VectorSubcoreMesh(
    core_axis_name="c", subcore_axis_name="s", num_cores=2, num_subcores=16
)
_SC_PARAMS = pltpu.CompilerParams(use_tc_tiling_on_sc=False)


def _deg_call(sblk, rblk, z16, ones16, kg=2):
    nbt = EPAD // EB
    nb = nbt // 32
    ng = nb // kg
    assert ng % 2 == 0 and ng >= 4

    @functools.partial(
        pl.kernel,
        out_type=jax.ShapeDtypeStruct((4, NPAD, 16), jnp.float32),
        mesh=_MESH,
        compiler_params=_SC_PARAMS,
        scratch_types=[
            pltpu.VMEM((2, kg, EB), jnp.int32),
            pltpu.VMEM((2, kg, EB), jnp.int32),
            pltpu.VMEM((2, kg, EB), jnp.int32),
            pltpu.VMEM((2, kg, EB), jnp.int32),
            pltpu.VMEM((EB, 16), jnp.float32),
            pltpu.VMEM_SHARED((NPAD, 16), jnp.float32),
            pltpu.VMEM_SHARED((NPAD, 16), jnp.float32),
            pltpu.SemaphoreType.DMA((2,)),
            pltpu.SemaphoreType.DMA((2,)),
            pltpu.SemaphoreType.DMA((2, kg)),
            pltpu.SemaphoreType.DMA((2, kg)),
        ],
    )
    def deg_kernel(s_hbm, r_hbm, z_hbm, ones_hbm, out_hbm,
                   slabs_v, slabr_v, sidx_v, ridx_v, ones_v,
                   accs_sh, accr_sh, isems, isemr, s1sem, s2sem):
        core = lax.axis_index("c")
        sub = lax.axis_index("s")
        base_row = sub * S
        pltpu.sync_copy(ones_hbm, ones_v)
        pltpu.sync_copy(z_hbm.at[pl.ds(base_row, S)],
                        accs_sh.at[pl.ds(base_row, S)])
        pltpu.sync_copy(z_hbm.at[pl.ds(base_row, S)],
                        accr_sh.at[pl.ds(base_row, S)])
        plsc.subcore_barrier()
        gbase = (core * 16 + sub) * ng

        def slab_copy_s(gblk, p):
            return pltpu.make_async_copy(
                s_hbm.at[pl.ds(gblk * kg, kg)], slabs_v.at[p], isems.at[p])

        def slab_copy_r(gblk, p):
            return pltpu.make_async_copy(
                r_hbm.at[pl.ds(gblk * kg, kg)], slabr_v.at[p], isemr.at[p])

        def scat1(p, k):
            return pltpu.make_async_copy(
                ones_v, accs_sh.at[sidx_v.at[p, k]], s1sem.at[p, k])

        def scat2(p, k):
            return pltpu.make_async_copy(
                ones_v, accr_sh.at[ridx_v.at[p, k]], s2sem.at[p, k])

        def run_group(g, p):
            slab_copy_s(gbase + g, p).wait()
            slab_copy_r(gbase + g, p).wait()
            for k in range(kg):
                @pl.when(g >= 2)
                def _():
                    scat1(p, k).wait()
                    scat2(p, k).wait()
                for v in range(EB // 16):
                    sl = pl.ds(v * 16, 16)
                    sidx_v[p, k, sl] = slabs_v[p, k, sl]
                    ridx_v[p, k, sl] = slabr_v[p, k, sl]
                pltpu.async_copy(ones_v, accs_sh.at[sidx_v.at[p, k]],
                                 s1sem.at[p, k], add=True)
                pltpu.async_copy(ones_v, accr_sh.at[ridx_v.at[p, k]],
                                 s2sem.at[p, k], add=True)
            @pl.when(g + 1 < ng)
            def _():
                slab_copy_s(gbase + g + 1, 1 - p).start()
                slab_copy_r(gbase + g + 1, 1 - p).start()

        slab_copy_s(gbase, 0).start()
        slab_copy_r(gbase, 0).start()

        def pair(t, carry):
            run_group(2 * t, 0)
            run_group(2 * t + 1, 1)
            return carry

        lax.fori_loop(0, ng // 2, pair, 0)
        for p in range(2):
            for k in range(kg):
                scat1(p, k).wait()
                scat2(p, k).wait()
        plsc.subcore_barrier()
        pltpu.sync_copy(accs_sh.at[pl.ds(base_row, S)],
                        out_hbm.at[core, pl.ds(base_row, S)])
        pltpu.sync_copy(accr_sh.at[pl.ds(base_row, S)],
                        out_hbm.at[core + 2, pl.ds(base_row, S)])

    return deg_kernel(sblk, rblk, z16, ones16)


def _prop_call(h_flat, sblk, rblk, z, ncc, nes, dc, kg):
    nunits = ncc * nes
    upc = nunits // 2
    nbt = EPAD // EB
    nb = nbt // (nes * 16)
    ng = nb // kg
    assert ng % 2 == 0 and ng >= 4

    @functools.partial(
        pl.kernel,
        out_type=jax.ShapeDtypeStruct((nunits, NPAD, dc), jnp.float32),
        mesh=_MESH,
        compiler_params=_SC_PARAMS,
        scratch_types=[
            pltpu.VMEM((2, kg, EB), jnp.int32),
            pltpu.VMEM((2, kg, EB), jnp.int32),
            pltpu.VMEM((2, kg, EB), jnp.int32),
            pltpu.VMEM((2, kg, EB), jnp.int32),
            pltpu.VMEM((2, kg, EB, dc), jnp.float32),
            pltpu.VMEM_SHARED((NPAD, dc), jnp.float32),
            pltpu.SemaphoreType.DMA((2,)),
            pltpu.SemaphoreType.DMA((2,)),
            pltpu.SemaphoreType.DMA((2, kg)),
            pltpu.SemaphoreType.DMA((2, kg)),
        ],
    )
    def prop_kernel(h_hbm, s_hbm, r_hbm, z_hbm, out_hbm,
                    slabs_v, slabr_v, sidx2_v, ridx_v, rows_v, acc_sh,
                    isems, isemr, gsem, ssem):
        core = lax.axis_index("c")
        sub = lax.axis_index("s")
        base_row = sub * S

        def slab_copy_s(gblk, p):
            return pltpu.make_async_copy(
                s_hbm.at[pl.ds(gblk * kg, kg)], slabs_v.at[p], isems.at[p])

        def slab_copy_r(gblk, p):
            return pltpu.make_async_copy(
                r_hbm.at[pl.ds(gblk * kg, kg)], slabr_v.at[p], isemr.at[p])

        def gather_copy(p, k):
            return pltpu.make_async_copy(
                h_hbm.at[sidx2_v.at[p, k]], rows_v.at[p, k], gsem.at[p, k])

        def scatter_copy(p, k):
            return pltpu.make_async_copy(
                rows_v.at[p, k], acc_sh.at[ridx_v.at[p, k]], ssem.at[p, k])

        for j in range(upc):
            u = 2 * j + core
            if nes == 1:
                cc, es = u, 0
            else:
                cc, es = 0, u
            pltpu.sync_copy(z_hbm.at[pl.ds(base_row, S)],
                            acc_sh.at[pl.ds(base_row, S)])
            plsc.subcore_barrier()
            gbase = es * (nbt // (nes * kg)) + sub * ng

            def run_group(g, p):
                slab_copy_s(gbase + g, p).wait()
                slab_copy_r(gbase + g, p).wait()
                for k in range(kg):
                    @pl.when(g >= 2)
                    def _():
                        scatter_copy(p, k).wait()
                    for v in range(EB // 16):
                        sl = pl.ds(v * 16, 16)
                        sidx2_v[p, k, sl] = slabs_v[p, k, sl] * ncc + cc
                        ridx_v[p, k, sl] = slabr_v[p, k, sl]
                    pltpu.async_copy(
                        h_hbm.at[sidx2_v.at[p, k]], rows_v.at[p, k],
                        gsem.at[p, k])
                @pl.when(g + 1 < ng)
                def _():
                    slab_copy_s(gbase + g + 1, 1 - p).start()
                    slab_copy_r(gbase + g + 1, 1 - p).start()
                for k in range(kg):
                    gather_copy(p, k).wait()
                    pltpu.async_copy(
                        rows_v.at[p, k], acc_sh.at[ridx_v.at[p, k]],
                        ssem.at[p, k], add=True)

            slab_copy_s(gbase, 0).start()
            slab_copy_r(gbase, 0).start()

            def pair(t, carry):
                run_group(2 * t, 0)
                run_group(2 * t + 1, 1)
                return carry

            lax.fori_loop(0, ng // 2, pair, 0)
            for p in range(2):
                for k in range(kg):
                    scatter_copy(p, k).wait()
            plsc.subcore_barrier()
            pltpu.sync_copy(acc_sh.at[pl.ds(base_row, S)],
                            out_hbm.at[u, pl.ds(base_row, S)])

    return prop_kernel(h_flat, sblk, rblk, z)


def _tc_layer(prev_ch, degs, wmat, bvec, relu_in):
    ncc_in, _, dc_in = prev_ch.shape
    din, dout = wmat.shape

    def body(prev_ref, degs_ref, w_ref, b_ref, out_ref):
        i = pl.program_id(0)
        rows = i * BLK + lax.broadcasted_iota(jnp.int32, (BLK, 16), 0)
        dego = degs_ref[0] + degs_ref[1]
        inv_out = jnp.where(
            rows < N, lax.rsqrt(jnp.maximum(dego, 1.0)), 0.0)[:, 0:1]
        if relu_in:
            degi = degs_ref[2] + degs_ref[3]
            inv_in = lax.rsqrt(jnp.maximum(degi, 1.0))[:, 0:1]
        h = jnp.zeros((BLK, dout), jnp.float32)
        for c in range(ncc_in):
            t = prev_ref[c]
            if relu_in:
                t = jnp.maximum(t * inv_in, 0.0)
            h = h + jnp.dot(t, w_ref[c * dc_in:(c + 1) * dc_in, :],
                            preferred_element_type=jnp.float32)
        out_ref[...] = (h + b_ref[...]) * inv_out

    return pl.pallas_call(
        body,
        grid=(NBLK,),
        in_specs=[
            pl.BlockSpec((ncc_in, BLK, dc_in), lambda i: (0, i, 0)),
            pl.BlockSpec((4, BLK, 16), lambda i: (0, i, 0)),
            pl.BlockSpec((din, dout), lambda i: (0, 0)),
            pl.BlockSpec((1, dout), lambda i: (0, 0)),
        ],
        out_specs=pl.BlockSpec((BLK, dout), lambda i: (i, 0)),
        out_shape=jax.ShapeDtypeStruct((NPAD, dout), jnp.float32),
    )(prev_ch, degs, wmat, bvec)


def _tc_readout(parts, degs, batcht):
    def body(parts_ref, degs_ref, batch_ref, out_ref):
        i = pl.program_id(0)

        @pl.when(i == 0)
        def _():
            out_ref[...] = jnp.zeros_like(out_ref)

        degi = degs_ref[2] + degs_ref[3]
        inv_in = lax.rsqrt(jnp.maximum(degi, 1.0))[:, 0:1]
        h = (parts_ref[0] + parts_ref[1]) * inv_in
        brow = batch_ref[...][0:1, :]
        gids = lax.broadcasted_iota(jnp.int32, (G, BLK), 0)
        mask_t = jnp.where(brow == gids, 1.0, 0.0)
        out_ref[...] += jnp.dot(mask_t, h,
                                preferred_element_type=jnp.float32)

    return pl.pallas_call(
        body,
        grid=(NBLK,),
        in_specs=[
            pl.BlockSpec((2, BLK, 16), lambda i: (0, i, 0)),
            pl.BlockSpec((4, BLK, 16), lambda i: (0, i, 0)),
            pl.BlockSpec((8, BLK), lambda i: (0, i)),
        ],
        out_specs=pl.BlockSpec((G, 16), lambda i: (0, 0)),
        out_shape=jax.ShapeDtypeStruct((G, 16), jnp.float32),
    )(parts, degs, batcht)


def kernel(x, senders, receivers, batch, n_node, num_graphs,
           W1, b1, W2, b2, W3, b3):
    xp = jnp.pad(x, ((0, NPAD - N), (0, 16 - x.shape[1])))
    sp = jnp.pad(senders, (0, EPAD - E), constant_values=N)
    rp = jnp.pad(receivers, (0, EPAD - E), constant_values=N)
    bpt = jnp.broadcast_to(
        jnp.pad(batch, (0, NPAD - N)).reshape(1, NPAD), (8, NPAD))
    z32 = jnp.zeros((NPAD, 32), jnp.float32)
    z16 = jnp.zeros((NPAD, 16), jnp.float32)
    ones16 = jnp.ones((EB, 16), jnp.float32)
    w1p = jnp.pad(W1, ((0, 16 - W1.shape[0]), (0, 0)))
    w3p = jnp.pad(W3, ((0, 0), (0, 16 - W3.shape[1])))
    b1r = b1.reshape(1, -1)
    b2r = b2.reshape(1, -1)
    b3r = jnp.pad(b3, (0, 16 - b3.shape[0])).reshape(1, -1)

    sblk = sp.reshape(-1, EB)
    rblk = rp.reshape(-1, EB)

    degs = _deg_call(sblk, rblk, z16, ones16)

    h1 = _tc_layer(xp.reshape(1, NPAD, 16), degs, w1p, b1r, False)
    a1 = _prop_call(h1.reshape(2 * NPAD, 32), sblk, rblk, z32, 2, 1, 32, 2)
    h2 = _tc_layer(a1, degs, W2, b2r, True)
    a2 = _prop_call(h2.reshape(4 * NPAD, 32), sblk, rblk, z32, 4, 1, 32, 2)
    h3 = _tc_layer(a2, degs, w3p, b3r, True)
    a3 = _prop_call(h3, sblk, rblk, z16, 1, 2, 16, 2)
    out = _tc_readout(a3, degs, bpt)
    return out[:, :10]


# speedup vs baseline: 1.2392x; 1.0749x over previous
import functools

import jax
import jax.numpy as jnp
from jax import lax
from jax.experimental import pallas as pl
from jax.experimental.pallas import tpu as pltpu
from jax.experimental.pallas import tpu_sc as plsc

N = 50000
NPAD = 50048
S = NPAD // 16
E = 800000
EPAD = 802816
EB = 128
G = 512
BLK = 2944
NBLK = NPAD // BLK

_MESH = plsc.VectorSubcoreMesh(
    core_axis_name="c", subcore_axis_name="s", num_cores=2, num_subcores=16
)
_SC_PARAMS = pltpu.CompilerParams(use_tc_tiling_on_sc=False)


def _deg_call(sblk, rblk, z16, ones16, kg=2):
    nbt = EPAD // EB
    nb = nbt // 32
    ng = nb // kg
    assert ng % 2 == 0 and ng >= 4

    @functools.partial(
        pl.kernel,
        out_type=jax.ShapeDtypeStruct((4, NPAD, 16), jnp.float32),
        mesh=_MESH,
        compiler_params=_SC_PARAMS,
        scratch_types=[
            pltpu.VMEM((2, kg, EB), jnp.int32),
            pltpu.VMEM((2, kg, EB), jnp.int32),
            pltpu.VMEM((2, kg, EB), jnp.int32),
            pltpu.VMEM((2, kg, EB), jnp.int32),
            pltpu.VMEM((EB, 16), jnp.float32),
            pltpu.VMEM_SHARED((NPAD, 16), jnp.float32),
            pltpu.VMEM_SHARED((NPAD, 16), jnp.float32),
            pltpu.SemaphoreType.DMA((2,)),
            pltpu.SemaphoreType.DMA((2,)),
            pltpu.SemaphoreType.DMA((2, kg)),
            pltpu.SemaphoreType.DMA((2, kg)),
        ],
    )
    def deg_kernel(s_hbm, r_hbm, z_hbm, ones_hbm, out_hbm,
                   slabs_v, slabr_v, sidx_v, ridx_v, ones_v,
                   accs_sh, accr_sh, isems, isemr, s1sem, s2sem):
        core = lax.axis_index("c")
        sub = lax.axis_index("s")
        base_row = sub * S
        pltpu.sync_copy(ones_hbm, ones_v)
        pltpu.sync_copy(z_hbm.at[pl.ds(base_row, S)],
                        accs_sh.at[pl.ds(base_row, S)])
        pltpu.sync_copy(z_hbm.at[pl.ds(base_row, S)],
                        accr_sh.at[pl.ds(base_row, S)])
        plsc.subcore_barrier()
        gbase = (core * 16 + sub) * ng

        def slab_copy_s(gblk, p):
            return pltpu.make_async_copy(
                s_hbm.at[pl.ds(gblk * kg, kg)], slabs_v.at[p], isems.at[p])

        def slab_copy_r(gblk, p):
            return pltpu.make_async_copy(
                r_hbm.at[pl.ds(gblk * kg, kg)], slabr_v.at[p], isemr.at[p])

        def scat1(p, k):
            return pltpu.make_async_copy(
                ones_v, accs_sh.at[sidx_v.at[p, k]], s1sem.at[p, k])

        def scat2(p, k):
            return pltpu.make_async_copy(
                ones_v, accr_sh.at[ridx_v.at[p, k]], s2sem.at[p, k])

        def run_group(g, p):
            slab_copy_s(gbase + g, p).wait()
            slab_copy_r(gbase + g, p).wait()
            for k in range(kg):
                @pl.when(g >= 2)
                def _():
                    scat1(p, k).wait()
                    scat2(p, k).wait()
                for v in range(EB // 16):
                    sl = pl.ds(v * 16, 16)
                    sidx_v[p, k, sl] = slabs_v[p, k, sl]
                    ridx_v[p, k, sl] = slabr_v[p, k, sl]
                pltpu.async_copy(ones_v, accs_sh.at[sidx_v.at[p, k]],
                                 s1sem.at[p, k], add=True)
                pltpu.async_copy(ones_v, accr_sh.at[ridx_v.at[p, k]],
                                 s2sem.at[p, k], add=True)
            @pl.when(g + 1 < ng)
            def _():
                slab_copy_s(gbase + g + 1, 1 - p).start()
                slab_copy_r(gbase + g + 1, 1 - p).start()

        slab_copy_s(gbase, 0).start()
        slab_copy_r(gbase, 0).start()

        def pair(t, carry):
            run_group(2 * t, 0)
            run_group(2 * t + 1, 1)
            return carry

        lax.fori_loop(0, ng // 2, pair, 0)
        for p in range(2):
            for k in range(kg):
                scat1(p, k).wait()
                scat2(p, k).wait()
        plsc.subcore_barrier()
        pltpu.sync_copy(accs_sh.at[pl.ds(base_row, S)],
                        out_hbm.at[core, pl.ds(base_row, S)])
        pltpu.sync_copy(accr_sh.at[pl.ds(base_row, S)],
                        out_hbm.at[core + 2, pl.ds(base_row, S)])

    return deg_kernel(sblk, rblk, z16, ones16)


def _prop_call(h_flat, sblk, rblk, z, ncc, nes, dc, kg):
    nunits = ncc * nes
    upc = nunits // 2
    nbt = EPAD // EB
    nb = nbt // (nes * 16)
    ng = nb // kg
    assert ng % 2 == 0 and ng >= 4

    @functools.partial(
        pl.kernel,
        out_type=jax.ShapeDtypeStruct((nunits, NPAD, dc), jnp.float32),
        mesh=_MESH,
        compiler_params=_SC_PARAMS,
        scratch_types=[
            pltpu.VMEM((2, kg, EB), jnp.int32),
            pltpu.VMEM((2, kg, EB), jnp.int32),
            pltpu.VMEM((2, kg, EB), jnp.int32),
            pltpu.VMEM((2, kg, EB), jnp.int32),
            pltpu.VMEM((2, kg, EB, dc), jnp.float32),
            pltpu.VMEM_SHARED((NPAD, dc), jnp.float32),
            pltpu.SemaphoreType.DMA((2,)),
            pltpu.SemaphoreType.DMA((2,)),
            pltpu.SemaphoreType.DMA((2, kg)),
            pltpu.SemaphoreType.DMA((2, kg)),
        ],
    )
    def prop_kernel(h_hbm, s_hbm, r_hbm, z_hbm, out_hbm,
                    slabs_v, slabr_v, sidx2_v, ridx_v, rows_v, acc_sh,
                    isems, isemr, gsem, ssem):
        core = lax.axis_index("c")
        sub = lax.axis_index("s")
        base_row = sub * S

        def slab_copy_s(gblk, p):
            return pltpu.make_async_copy(
                s_hbm.at[pl.ds(gblk * kg, kg)], slabs_v.at[p], isems.at[p])

        def slab_copy_r(gblk, p):
            return pltpu.make_async_copy(
                r_hbm.at[pl.ds(gblk * kg, kg)], slabr_v.at[p], isemr.at[p])

        def gather_copy(p, k):
            return pltpu.make_async_copy(
                h_hbm.at[sidx2_v.at[p, k]], rows_v.at[p, k], gsem.at[p, k])

        def scatter_copy(p, k):
            return pltpu.make_async_copy(
                rows_v.at[p, k], acc_sh.at[ridx_v.at[p, k]], ssem.at[p, k])

        for j in range(upc):
            u = 2 * j + core
            if nes == 1:
                cc, es = u, 0
            else:
                cc, es = 0, u
            pltpu.sync_copy(z_hbm.at[pl.ds(base_row, S)],
                            acc_sh.at[pl.ds(base_row, S)])
            plsc.subcore_barrier()
            gbase = es * (nbt // (nes * kg)) + sub * ng

            def run_group(g, p):
                slab_copy_s(gbase + g, p).wait()
                slab_copy_r(gbase + g, p).wait()
                for k in range(kg):
                    @pl.when(g >= 2)
                    def _():
                        scatter_copy(p, k).wait()
                    for v in range(EB // 16):
                        sl = pl.ds(v * 16, 16)
                        sidx2_v[p, k, sl] = slabs_v[p, k, sl] * ncc + cc
                        ridx_v[p, k, sl] = slabr_v[p, k, sl]
                    pltpu.async_copy(
                        h_hbm.at[sidx2_v.at[p, k]], rows_v.at[p, k],
                        gsem.at[p, k])
                @pl.when(g + 1 < ng)
                def _():
                    slab_copy_s(gbase + g + 1, 1 - p).start()
                    slab_copy_r(gbase + g + 1, 1 - p).start()
                for k in range(kg):
                    @pl.when(g >= 1)
                    def _():
                        gather_copy(1 - p, k).wait()
                        pltpu.async_copy(
                            rows_v.at[1 - p, k],
                            acc_sh.at[ridx_v.at[1 - p, k]],
                            ssem.at[1 - p, k], add=True)

            slab_copy_s(gbase, 0).start()
            slab_copy_r(gbase, 0).start()

            def pair(t, carry):
                run_group(2 * t, 0)
                run_group(2 * t + 1, 1)
                return carry

            lax.fori_loop(0, ng // 2, pair, 0)
            for k in range(kg):
                gather_copy(1, k).wait()
                pltpu.async_copy(
                    rows_v.at[1, k], acc_sh.at[ridx_v.at[1, k]],
                    ssem.at[1, k], add=True)
            for p in range(2):
                for k in range(kg):
                    scatter_copy(p, k).wait()
            plsc.subcore_barrier()
            pltpu.sync_copy(acc_sh.at[pl.ds(base_row, S)],
                            out_hbm.at[u, pl.ds(base_row, S)])

    return prop_kernel(h_flat, sblk, rblk, z)


def _tc_layer(prev_ch, degs, wmat, bvec, relu_in):
    ncc_in, _, dc_in = prev_ch.shape
    din, dout = wmat.shape

    def body(prev_ref, degs_ref, w_ref, b_ref, out_ref):
        i = pl.program_id(0)
        rows = i * BLK + lax.broadcasted_iota(jnp.int32, (BLK, 16), 0)
        dego = degs_ref[0] + degs_ref[1]
        inv_out = jnp.where(
            rows < N, lax.rsqrt(jnp.maximum(dego, 1.0)), 0.0)[:, 0:1]
        if relu_in:
            degi = degs_ref[2] + degs_ref[3]
            inv_in = lax.rsqrt(jnp.maximum(degi, 1.0))[:, 0:1]
        h = jnp.zeros((BLK, dout), jnp.float32)
        for c in range(ncc_in):
            t = prev_ref[c]
            if relu_in:
                t = jnp.maximum(t * inv_in, 0.0)
            h = h + jnp.dot(t, w_ref[c * dc_in:(c + 1) * dc_in, :],
                            preferred_element_type=jnp.float32)
        out_ref[...] = (h + b_ref[...]) * inv_out

    return pl.pallas_call(
        body,
        grid=(NBLK,),
        in_specs=[
            pl.BlockSpec((ncc_in, BLK, dc_in), lambda i: (0, i, 0)),
            pl.BlockSpec((4, BLK, 16), lambda i: (0, i, 0)),
            pl.BlockSpec((din, dout), lambda i: (0, 0)),
            pl.BlockSpec((1, dout), lambda i: (0, 0)),
        ],
        out_specs=pl.BlockSpec((BLK, dout), lambda i: (i, 0)),
        out_shape=jax.ShapeDtypeStruct((NPAD, dout), jnp.float32),
    )(prev_ch, degs, wmat, bvec)


def _tc_readout(parts, degs, batcht):
    def body(parts_ref, degs_ref, batch_ref, out_ref):
        i = pl.program_id(0)

        @pl.when(i == 0)
        def _():
            out_ref[...] = jnp.zeros_like(out_ref)

        degi = degs_ref[2] + degs_ref[3]
        inv_in = lax.rsqrt(jnp.maximum(degi, 1.0))[:, 0:1]
        h = (parts_ref[0] + parts_ref[1]) * inv_in
        brow = batch_ref[...][0:1, :]
        gids = lax.broadcasted_iota(jnp.int32, (G, BLK), 0)
        mask_t = jnp.where(brow == gids, 1.0, 0.0)
        out_ref[...] += jnp.dot(mask_t, h,
                                preferred_element_type=jnp.float32)

    return pl.pallas_call(
        body,
        grid=(NBLK,),
        in_specs=[
            pl.BlockSpec((2, BLK, 16), lambda i: (0, i, 0)),
            pl.BlockSpec((4, BLK, 16), lambda i: (0, i, 0)),
            pl.BlockSpec((8, BLK), lambda i: (0, i)),
        ],
        out_specs=pl.BlockSpec((G, 16), lambda i: (0, 0)),
        out_shape=jax.ShapeDtypeStruct((G, 16), jnp.float32),
    )(parts, degs, batcht)


def kernel(x, senders, receivers, batch, n_node, num_graphs,
           W1, b1, W2, b2, W3, b3):
    xp = jnp.pad(x, ((0, NPAD - N), (0, 16 - x.shape[1])))
    sp = jnp.pad(senders, (0, EPAD - E), constant_values=N)
    rp = jnp.pad(receivers, (0, EPAD - E), constant_values=N)
    bpt = jnp.broadcast_to(
        jnp.pad(batch, (0, NPAD - N)).reshape(1, NPAD), (8, NPAD))
    z32 = jnp.zeros((NPAD, 32), jnp.float32)
    z16 = jnp.zeros((NPAD, 16), jnp.float32)
    ones16 = jnp.ones((EB, 16), jnp.float32)
    w1p = jnp.pad(W1, ((0, 16 - W1.shape[0]), (0, 0)))
    w3p = jnp.pad(W3, ((0, 0), (0, 16 - W3.shape[1])))
    b1r = b1.reshape(1, -1)
    b2r = b2.reshape(1, -1)
    b3r = jnp.pad(b3, (0, 16 - b3.shape[0])).reshape(1, -1)

    sblk = sp.reshape(-1, EB)
    rblk = rp.reshape(-1, EB)

    degs = _deg_call(sblk, rblk, z16, ones16)

    h1 = _tc_layer(xp.reshape(1, NPAD, 16), degs, w1p, b1r, False)
    a1 = _prop_call(h1.reshape(2 * NPAD, 32), sblk, rblk, z32, 2, 1, 32, 2)
    h2 = _tc_layer(a1, degs, W2, b2r, True)
    a2 = _prop_call(h2.reshape(4 * NPAD, 32), sblk, rblk, z32, 4, 1, 32, 2)
    h3 = _tc_layer(a2, degs, w3p, b3r, True)
    a3 = _prop_call(h3, sblk, rblk, z16, 1, 2, 16, 2)
    out = _tc_readout(a3, degs, bpt)
    return out[:, :10]
